# baseline (speedup 1.0000x reference)
import jax
import jax.numpy as jnp
from jax import lax
from jax.experimental import pallas as pl
from jax.experimental.pallas import tpu as pltpu
from jax.experimental.pallas import tpu_sc as plsc

B = 16384
C = 1000
L = 16
NC = 2
NS = 16
NW = NC * NS
BPW = B // NW
NCHUNK = 4
CHUNK = BPW // NCHUNK
NV = BPW // L


def _sc_gather_loss(logits_hbm, labels_hbm, out_hbm, labels_v, idx_v,
                    vals_v, acc_v, sem):
    wid = lax.axis_index("s") * NC + lax.axis_index("c")
    base = wid * BPW
    pltpu.sync_copy(labels_hbm.at[pl.ds(base, BPW)], labels_v)

    iota = lax.iota(jnp.int32, L)
    for j in range(NV):
        lbl = labels_v[pl.ds(j * L, L)]
        flat = (base + j * L + iota) * C + lbl
        idx_v[j // (CHUNK // L), pl.ds((j % (CHUNK // L)) * L, L)] = flat

    copies = [
        pltpu.async_copy(logits_hbm.at[idx_v.at[k]],
                         vals_v.at[pl.ds(k * CHUNK, CHUNK)], sem)
        for k in range(NCHUNK)
    ]
    for c in copies:
        c.wait()

    acc = jnp.zeros((L,), jnp.float32)
    for j in range(NV):
        acc = acc + vals_v[pl.ds(j * L, L)]

    acc_v[...] = acc * (-10.0 / B)
    pltpu.sync_copy(acc_v, out_hbm.at[wid])


@jax.jit
def kernel(logits, labels):
    flat_logits = logits.reshape(B * C)
    labels32 = labels.astype(jnp.int32)
    partials = pl.kernel(
        _sc_gather_loss,
        out_type=jax.ShapeDtypeStruct((NW, L), jnp.float32),
        mesh=plsc.VectorSubcoreMesh(core_axis_name="c", subcore_axis_name="s"),
        scratch_types=[
            pltpu.VMEM((BPW,), jnp.int32),
            pltpu.VMEM((NCHUNK, CHUNK), jnp.int32),
            pltpu.VMEM((BPW,), jnp.float32),
            pltpu.VMEM((L,), jnp.float32),
            pltpu.SemaphoreType.DMA,
        ],
    )(flat_logits, labels32)
    return 10.0 + jnp.sum(partials)


# speedup vs baseline: 1.2861x; 1.2861x over previous
import jax
import jax.numpy as jnp
from jax import lax
from jax.experimental import pallas as pl
from jax.experimental.pallas import tpu as pltpu

B = 16384
C = 1000
RB = 256
NBLK = B // RB


def _tc_select_reduce(logits_ref, labels_ref, out_ref):
    pid = pl.program_id(0)

    @pl.when(pid == 0)
    def _init():
        out_ref[0, 0] = 0.0

    lbl = labels_ref[0, 0, :]
    col = lax.broadcasted_iota(jnp.int32, (RB, C), 1)
    sel = jnp.where(col == lbl[:, None], logits_ref[...], 0.0)
    out_ref[0, 0] += jnp.sum(sel)


@jax.jit
def kernel(logits, labels):
    labels3 = labels.astype(jnp.int32).reshape(NBLK, 1, RB)
    total = pl.pallas_call(
        _tc_select_reduce,
        grid=(NBLK,),
        in_specs=[
            pl.BlockSpec((RB, C), lambda i: (i, 0)),
            pl.BlockSpec((1, 1, RB), lambda i: (i, 0, 0)),
        ],
        out_specs=pl.BlockSpec(memory_space=pltpu.SMEM),
        out_shape=jax.ShapeDtypeStruct((1, 1), jnp.float32),
    )(logits, labels3)
    return 10.0 * (1.0 - total[0, 0] / B)


# speedup vs baseline: 1.3668x; 1.0627x over previous
import jax
import jax.numpy as jnp
from jax import lax
from jax.experimental import pallas as pl
from jax.experimental.pallas import tpu as pltpu

B = 16384
C = 1000
RB = 256
NBLK = B // RB


def _tc_select_reduce(logits_ref, labels_ref, out_ref, acc_ref):
    pid = pl.program_id(0)

    @pl.when(pid == 0)
    def _init():
        acc_ref[...] = jnp.zeros_like(acc_ref)

    lbl = labels_ref[0, 0, :]
    col = lax.broadcasted_iota(jnp.int32, (RB, C), 1)
    sel = jnp.where(col == lbl[:, None], logits_ref[...], 0.0)
    acc_ref[...] += jnp.sum(sel, axis=0, keepdims=True)

    @pl.when(pid == NBLK - 1)
    def _final():
        out_ref[0, 0] = 10.0 * (1.0 - jnp.sum(acc_ref[...]) / B)


@jax.jit
def kernel(logits, labels):
    labels3 = labels.astype(jnp.int32).reshape(NBLK, 1, RB)
    total = pl.pallas_call(
        _tc_select_reduce,
        grid=(NBLK,),
        in_specs=[
            pl.BlockSpec((RB, C), lambda i: (i, 0)),
            pl.BlockSpec((1, 1, RB), lambda i: (i, 0, 0)),
        ],
        out_specs=pl.BlockSpec(memory_space=pltpu.SMEM),
        out_shape=jax.ShapeDtypeStruct((1, 1), jnp.float32),
        scratch_shapes=[pltpu.VMEM((1, C), jnp.float32)],
    )(logits, labels3)
    return 10.0 * (1.0 - total[0, 0] / B)


# speedup vs baseline: 1.8036x; 1.3196x over previous
import jax
import jax.numpy as jnp
from jax import lax
from jax.experimental import pallas as pl
from jax.experimental.pallas import tpu as pltpu

B = 16384
C = 1000
RB = 1024
NBLK = B // RB


def _tc_select_reduce(logits_ref, labels_ref, out_ref, acc_ref):
    pid = pl.program_id(0)

    @pl.when(pid == 0)
    def _init():
        acc_ref[...] = jnp.zeros_like(acc_ref)

    lbl = labels_ref[0, 0, :]
    col = lax.broadcasted_iota(jnp.int32, (RB, C), 1)
    sel = jnp.where(col == lbl[:, None], logits_ref[...], 0.0)
    acc_ref[...] += jnp.sum(sel, axis=0, keepdims=True)

    @pl.when(pid == NBLK - 1)
    def _final():
        out_ref[0, 0] = 10.0 * (1.0 - jnp.sum(acc_ref[...]) / B)


@jax.jit
def kernel(logits, labels):
    labels3 = labels.astype(jnp.int32).reshape(NBLK, 1, RB)
    total = pl.pallas_call(
        _tc_select_reduce,
        grid=(NBLK,),
        in_specs=[
            pl.BlockSpec((RB, C), lambda i: (i, 0)),
            pl.BlockSpec((1, 1, RB), lambda i: (i, 0, 0)),
        ],
        out_specs=pl.BlockSpec(memory_space=pltpu.SMEM),
        out_shape=jax.ShapeDtypeStruct((1, 1), jnp.float32),
        scratch_shapes=[pltpu.VMEM((1, C), jnp.float32)],
    )(logits, labels3)
    return 10.0 * (1.0 - total[0, 0] / B)


# speedup vs baseline: 5.0436x; 2.7964x over previous
import jax
import jax.numpy as jnp
from jax import lax
from jax.experimental import pallas as pl
from jax.experimental.pallas import tpu as pltpu

B = 16384
C = 1000
CB = 40
NBLK = C // CB


def _tc_select_reduce(logits_ref, labels_ref, out_ref, acc_ref):
    pid = pl.program_id(0)

    @pl.when(pid == 0)
    def _init():
        acc_ref[...] = jnp.zeros_like(acc_ref)

    lbl = labels_ref[...]
    row = lax.broadcasted_iota(jnp.int32, (CB, B), 0) + pid * CB
    sel = jnp.where(row == lbl, logits_ref[...], 0.0)
    acc_ref[...] += jnp.sum(sel, axis=0, keepdims=True)

    @pl.when(pid == NBLK - 1)
    def _final():
        out_ref[0, 0] = 10.0 * (1.0 - jnp.sum(acc_ref[...]) / B)


@jax.jit
def kernel(logits, labels):
    logits_t = logits.T
    labels2 = labels.astype(jnp.int32).reshape(1, B)
    total = pl.pallas_call(
        _tc_select_reduce,
        grid=(NBLK,),
        in_specs=[
            pl.BlockSpec((CB, B), lambda c: (c, 0)),
            pl.BlockSpec((1, B), lambda c: (0, 0)),
        ],
        out_specs=pl.BlockSpec(memory_space=pltpu.SMEM),
        out_shape=jax.ShapeDtypeStruct((1, 1), jnp.float32),
        scratch_shapes=[pltpu.VMEM((1, B), jnp.float32)],
    )(logits_t, labels2)
    return total[0, 0]


# speedup vs baseline: 5.2758x; 1.0460x over previous
import jax
import jax.numpy as jnp
from jax import lax
from jax.experimental import pallas as pl
from jax.experimental.pallas import tpu as pltpu

B = 16384
C = 1000
CB = 40
NBLK = C // CB


def _tc_select_reduce(logits_ref, labels_ref, out_ref, acc_ref):
    pid = pl.program_id(0)

    @pl.when(pid == 0)
    def _init():
        acc_ref[...] = jnp.zeros_like(acc_ref)

    lbl = labels_ref[...]
    row = lax.broadcasted_iota(jnp.int32, (CB, B), 0) + pid * CB
    sel = jnp.where(row == lbl, logits_ref[...], 0.0)
    part = sel[0:8]
    for s in range(8, CB, 8):
        part = part + sel[s:s + 8]
    acc_ref[...] += part

    @pl.when(pid == NBLK - 1)
    def _final():
        out_ref[0, 0] = 10.0 * (1.0 - jnp.sum(acc_ref[...]) / B)


@jax.jit
def kernel(logits, labels):
    logits_t = logits.T
    labels2 = labels.astype(jnp.int32).reshape(1, B)
    total = pl.pallas_call(
        _tc_select_reduce,
        grid=(NBLK,),
        in_specs=[
            pl.BlockSpec((CB, B), lambda c: (c, 0)),
            pl.BlockSpec((1, B), lambda c: (0, 0)),
        ],
        out_specs=pl.BlockSpec(memory_space=pltpu.SMEM),
        out_shape=jax.ShapeDtypeStruct((1, 1), jnp.float32),
        scratch_shapes=[pltpu.VMEM((8, B), jnp.float32)],
    )(logits_t, labels2)
    return total[0, 0]


# speedup vs baseline: 6.2585x; 1.1863x over previous
import jax
import jax.numpy as jnp
from jax import lax
from jax.experimental import pallas as pl
from jax.experimental.pallas import tpu as pltpu
from jax.experimental.pallas import tpu_sc as plsc

B = 16384
C = 1000
L = 16
NC = 2
NS = 16
NW = NC * NS
BPW = B // NW
NCHUNK = 4
CHUNK = BPW // NCHUNK
NV = BPW // L


def _sc_gather_loss(logits_hbm, labels_hbm, out_hbm, labels_v, idx_v,
                    vals_v, acc_v, sem):
    wid = lax.axis_index("s") * NC + lax.axis_index("c")
    base = wid * BPW
    pltpu.sync_copy(labels_hbm.at[pl.ds(base, BPW)], labels_v)

    iota = lax.iota(jnp.int32, L)
    for j in range(NV):
        lbl = labels_v[pl.ds(j * L, L)]
        i_vec = base + j * L + iota
        w = (lax.shift_left(lax.shift_right_logical(lbl, 3), 17)
             + lax.shift_left(lax.shift_right_logical(i_vec, 7), 10)
             + lax.shift_left(jnp.bitwise_and(lbl, 7), 7)
             + jnp.bitwise_and(i_vec, 127))
        idx_v[j // (CHUNK // L), pl.ds((j % (CHUNK // L)) * L, L)] = w

    copies = [
        pltpu.async_copy(logits_hbm.at[idx_v.at[k]],
                         vals_v.at[pl.ds(k * CHUNK, CHUNK)], sem)
        for k in range(NCHUNK)
    ]
    for c in copies:
        c.wait()

    acc = jnp.zeros((L,), jnp.float32)
    for j in range(NV):
        acc = acc + vals_v[pl.ds(j * L, L)]

    acc_v[...] = acc * (-10.0 / B)
    pltpu.sync_copy(acc_v, out_hbm.at[wid])


@jax.jit
def kernel(logits, labels):
    flat_logits = (logits.T.reshape(C // 8, 8, B // 128, 128)
                   .transpose(0, 2, 1, 3).reshape(B * C))
    labels32 = labels.astype(jnp.int32)
    partials = pl.kernel(
        _sc_gather_loss,
        out_type=jax.ShapeDtypeStruct((NW, L), jnp.float32),
        mesh=plsc.VectorSubcoreMesh(core_axis_name="c", subcore_axis_name="s"),
        scratch_types=[
            pltpu.VMEM((BPW,), jnp.int32),
            pltpu.VMEM((NCHUNK, CHUNK), jnp.int32),
            pltpu.VMEM((BPW,), jnp.float32),
            pltpu.VMEM((L,), jnp.float32),
            pltpu.SemaphoreType.DMA,
        ],
    )(flat_logits, labels32)
    return 10.0 + jnp.sum(partials)
